# Initial kernel scaffold; baseline (speedup 1.0000x reference)
import jax
import jax.numpy as jnp
from jax.experimental import pallas as pl


def kernel(input, eidx, enorm, esgn):
    raise NotImplementedError("write your pallas kernel here")


# speedup vs baseline: 6.7143x; 6.7143x over previous
import functools

import jax
import jax.numpy as jnp
from jax import lax
from jax.experimental import pallas as pl
from jax.experimental.pallas import tpu as pltpu
from jax.experimental.pallas import tpu_sc as plsc

N_NODES = 10000
N_EDGES = 320000
D = 128
L = 16

NC = 2
NS = 16
NW = NC * NS
EPW = N_EDGES // NW
CHUNK = 80
SCH = 25
NSCH = EPW // (CHUNK * SCH)
WROWS = N_NODES // NS
WCH = 125

_BCAST_DNUMS = lax.GatherDimensionNumbers(
    offset_dims=(), collapsed_slice_dims=(0,), start_index_map=(0,))


def _bcast_lane(v, j):
  idx = jnp.full((L, 1), j, dtype=jnp.int32)
  return lax.gather(v, idx, _BCAST_DNUMS, (1,),
                    mode=lax.GatherScatterMode.PROMISE_IN_BOUNDS)


def _sc_body(x_hbm, sidx_hbm, tidx_hbm, en_hbm, es_hbm, out_hbm,
             acc_sh, sidx_v, tidx_v, en_v, es_v, rows_v):
  cid = lax.axis_index("c")
  sid = lax.axis_index("s")
  wid = cid * NS + sid

  zvec = jnp.zeros((L,), jnp.float32)

  def _zrow(i, _):
    for k in range(D // L):
      rows_v[i, pl.ds(k * L, L)] = zvec
    return 0

  lax.fori_loop(0, WCH, _zrow, 0)
  for r in range(WROWS // WCH):
    pltpu.sync_copy(rows_v, acc_sh.at[pl.ds(sid * WROWS + r * WCH, WCH)])

  plsc.subcore_barrier()

  def _super(s, _):
    pltpu.sync_copy(sidx_hbm.at[wid, s], sidx_v)
    pltpu.sync_copy(tidx_hbm.at[wid, s], tidx_v)
    pltpu.sync_copy(en_hbm.at[wid, s], en_v)
    pltpu.sync_copy(es_hbm.at[wid, s], es_v)

    def _chunk(c, _):
      pltpu.sync_copy(x_hbm.at[sidx_v.at[c]], rows_v.at[pl.ds(0, CHUNK)])

      def _group(g, _):
        w = en_v[c, pl.ds(g * L, L)] * es_v[c, pl.ds(g * L, L)]
        for j in range(L):
          e = g * L + j
          wj = _bcast_lane(w, j)
          for k in range(D // L):
            rows_v[e, pl.ds(k * L, L)] = rows_v[e, pl.ds(k * L, L)] * wj
        return 0

      lax.fori_loop(0, CHUNK // L, _group, 0)

      pltpu.sync_copy(rows_v.at[pl.ds(0, CHUNK)], acc_sh.at[tidx_v.at[c]],
                      add=True)
      return 0

    lax.fori_loop(0, SCH, _chunk, 0)
    return 0

  lax.fori_loop(0, NSCH, _super, 0)

  plsc.subcore_barrier()

  for r in range(WROWS // WCH):
    row0 = sid * WROWS + r * WCH
    pltpu.sync_copy(acc_sh.at[pl.ds(row0, WCH)], rows_v)
    pltpu.sync_copy(rows_v, out_hbm.at[cid, pl.ds(row0, WCH)])


_sc_kernel = functools.partial(
    pl.kernel,
    out_type=jax.ShapeDtypeStruct((NC, N_NODES, D), jnp.float32),
    mesh=plsc.VectorSubcoreMesh(core_axis_name="c", subcore_axis_name="s"),
    compiler_params=pltpu.CompilerParams(use_tc_tiling_on_sc=False),
    scratch_types=[
        pltpu.VMEM_SHARED((N_NODES, D), jnp.float32),
        pltpu.VMEM((SCH, CHUNK), jnp.int32),
        pltpu.VMEM((SCH, CHUNK), jnp.int32),
        pltpu.VMEM((SCH, CHUNK), jnp.float32),
        pltpu.VMEM((SCH, CHUNK), jnp.float32),
        pltpu.VMEM((WCH, D), jnp.float32),
    ],
)(_sc_body)


def _add_body(a_ref, o_ref):
  o_ref[...] = a_ref[0] + a_ref[1]


def _combine(partials):
  blk = N_NODES // 10
  return pl.pallas_call(
      _add_body,
      out_shape=jax.ShapeDtypeStruct((N_NODES, D), jnp.float32),
      grid=(N_NODES // blk,),
      in_specs=[pl.BlockSpec((NC, blk, D), lambda i: (0, i, 0))],
      out_specs=pl.BlockSpec((blk, D), lambda i: (i, 0)),
  )(partials)


def kernel(input, eidx, enorm, esgn):
  sidx = eidx[0].astype(jnp.int32).reshape(NW, NSCH, SCH, CHUNK)
  tidx = eidx[1].astype(jnp.int32).reshape(NW, NSCH, SCH, CHUNK)
  en = enorm.reshape(NW, NSCH, SCH, CHUNK)
  es = esgn.reshape(NW, NSCH, SCH, CHUNK)
  partials = _sc_kernel(input, sidx, tidx, en, es)
  return _combine(partials)


# speedup vs baseline: 9.2250x; 1.3739x over previous
import functools

import jax
import jax.numpy as jnp
from jax import lax
from jax.experimental import pallas as pl
from jax.experimental.pallas import tpu as pltpu
from jax.experimental.pallas import tpu_sc as plsc

N_NODES = 10000
N_EDGES = 320000
D = 128
L = 16

NC = 2
NS = 16
NW = NC * NS
EPW = N_EDGES // NW
CHUNK = 80
NCHUNK = EPW // CHUNK
SCH = 25
NSCH = NCHUNK // SCH
WROWS = N_NODES // NS

_BCAST_DNUMS = lax.GatherDimensionNumbers(
    offset_dims=(), collapsed_slice_dims=(0,), start_index_map=(0,))


def _bcast_lane(v, j):
  idx = jnp.full((L, 1), j, dtype=jnp.int32)
  return lax.gather(v, idx, _BCAST_DNUMS, (1,),
                    mode=lax.GatherScatterMode.PROMISE_IN_BOUNDS)


def _sc_body(x_hbm, sidx_hbm, tidx_hbm, en_hbm, es_hbm, out_hbm,
             acc_sh, sidx_v, tidx_v, en_v, es_v, rows_a, rows_b,
             gsem_a, gsem_b, ssem_a, ssem_b):
  cid = lax.axis_index("c")
  sid = lax.axis_index("s")
  wid = cid * NS + sid

  zvec = jnp.zeros((L,), jnp.float32)

  def _zrow(i, _):
    for k in range(D // L):
      rows_a[i, pl.ds(k * L, L)] = zvec
    return 0

  lax.fori_loop(0, CHUNK, _zrow, 0)
  for r in range(7):
    pltpu.sync_copy(rows_a, acc_sh.at[pl.ds(sid * WROWS + r * CHUNK, CHUNK)])
  pltpu.sync_copy(rows_a.at[pl.ds(0, WROWS - 7 * CHUNK)],
                  acc_sh.at[pl.ds(sid * WROWS + 7 * CHUNK,
                                  WROWS - 7 * CHUNK)])

  pltpu.sync_copy(sidx_hbm.at[wid], sidx_v)
  pltpu.sync_copy(tidx_hbm.at[wid], tidx_v)

  plsc.subcore_barrier()

  def _stage_w(c):
    @pl.when(lax.rem(c, SCH) == 0)
    def _():
      s = lax.div(c, SCH)
      pltpu.sync_copy(en_hbm.at[wid, s], en_v)
      pltpu.sync_copy(es_hbm.at[wid, s], es_v)

  def _gather_start(c, rows, sem):
    pltpu.async_copy(x_hbm.at[sidx_v.at[c]], rows, sem)

  def _gather_wait(c, rows, sem):
    pltpu.make_async_copy(x_hbm.at[sidx_v.at[c]], rows, sem).wait()

  def _scat_start(c, rows, sem):
    pltpu.async_copy(rows, acc_sh.at[tidx_v.at[c]], sem, add=True)

  def _scat_wait(c, rows, sem):
    pltpu.make_async_copy(rows, acc_sh.at[tidx_v.at[c]], sem).wait()

  def _scale(c, rows):
    c_l = lax.rem(c, SCH)

    def _group(g, _):
      w = en_v[c_l, pl.ds(g * L, L)] * es_v[c_l, pl.ds(g * L, L)]
      for j in range(L):
        e = g * L + j
        wj = _bcast_lane(w, j)
        for k in range(D // L):
          rows[e, pl.ds(k * L, L)] = rows[e, pl.ds(k * L, L)] * wj
      return 0

    lax.fori_loop(0, CHUNK // L, _group, 0)

  _gather_start(0, rows_a, gsem_a)

  def _pair(g, _):
    c0 = 2 * g
    c1 = c0 + 1
    _stage_w(c0)
    _gather_wait(c0, rows_a, gsem_a)

    @pl.when(g > 0)
    def _():
      _scat_wait(c1, rows_b, ssem_b)

    _gather_start(c1, rows_b, gsem_b)
    _scale(c0, rows_a)
    _scat_start(c0, rows_a, ssem_a)

    _stage_w(c1)
    _gather_wait(c1, rows_b, gsem_b)
    _scale(c1, rows_b)
    _scat_wait(c0, rows_a, ssem_a)
    _gather_start(c0 + 2, rows_a, gsem_a)
    _scat_start(c1, rows_b, ssem_b)
    return 0

  lax.fori_loop(0, (NCHUNK - 1) // 2, _pair, 0)

  ct = NCHUNK - 1
  _gather_wait(ct, rows_a, gsem_a)
  _scale(ct, rows_a)
  _scat_start(ct, rows_a, ssem_a)
  _scat_wait(ct, rows_a, ssem_a)
  _scat_wait(ct - 1, rows_b, ssem_b)

  plsc.subcore_barrier()

  for r in range(7):
    row0 = sid * WROWS + r * CHUNK
    pltpu.sync_copy(acc_sh.at[pl.ds(row0, CHUNK)], rows_a)
    pltpu.sync_copy(rows_a, out_hbm.at[cid, pl.ds(row0, CHUNK)])
  tail = WROWS - 7 * CHUNK
  row0 = sid * WROWS + 7 * CHUNK
  pltpu.sync_copy(acc_sh.at[pl.ds(row0, tail)], rows_a.at[pl.ds(0, tail)])
  pltpu.sync_copy(rows_a.at[pl.ds(0, tail)], out_hbm.at[cid, pl.ds(row0, tail)])


_sc_kernel = functools.partial(
    pl.kernel,
    out_type=jax.ShapeDtypeStruct((NC, N_NODES, D), jnp.float32),
    mesh=plsc.VectorSubcoreMesh(core_axis_name="c", subcore_axis_name="s"),
    compiler_params=pltpu.CompilerParams(use_tc_tiling_on_sc=False),
    scratch_types=[
        pltpu.VMEM_SHARED((N_NODES, D), jnp.float32),
        pltpu.VMEM((NCHUNK, CHUNK), jnp.int32),
        pltpu.VMEM((NCHUNK, CHUNK), jnp.int32),
        pltpu.VMEM((SCH, CHUNK), jnp.float32),
        pltpu.VMEM((SCH, CHUNK), jnp.float32),
        pltpu.VMEM((CHUNK, D), jnp.float32),
        pltpu.VMEM((CHUNK, D), jnp.float32),
        pltpu.SemaphoreType.DMA,
        pltpu.SemaphoreType.DMA,
        pltpu.SemaphoreType.DMA,
        pltpu.SemaphoreType.DMA,
    ],
)(_sc_body)


def _add_body(a_ref, o_ref):
  o_ref[...] = a_ref[0] + a_ref[1]


def _combine(partials):
  blk = N_NODES // 10
  return pl.pallas_call(
      _add_body,
      out_shape=jax.ShapeDtypeStruct((N_NODES, D), jnp.float32),
      grid=(N_NODES // blk,),
      in_specs=[pl.BlockSpec((NC, blk, D), lambda i: (0, i, 0))],
      out_specs=pl.BlockSpec((blk, D), lambda i: (i, 0)),
  )(partials)


def kernel(input, eidx, enorm, esgn):
  sidx = eidx[0].astype(jnp.int32).reshape(NW, NCHUNK, CHUNK)
  tidx = eidx[1].astype(jnp.int32).reshape(NW, NCHUNK, CHUNK)
  en = enorm.reshape(NW, NSCH, SCH, CHUNK)
  es = esgn.reshape(NW, NSCH, SCH, CHUNK)
  partials = _sc_kernel(input, sidx, tidx, en, es)
  return _combine(partials)


# speedup vs baseline: 9.2339x; 1.0010x over previous
import functools

import jax
import jax.numpy as jnp
from jax import lax
from jax.experimental import pallas as pl
from jax.experimental.pallas import tpu as pltpu
from jax.experimental.pallas import tpu_sc as plsc

N_NODES = 10000
N_EDGES = 320000
D = 128
L = 16

NC = 2
NS = 16
NW = NC * NS
EPW = N_EDGES // NW
CHUNK = 80
NCHUNK = EPW // CHUNK
SCH = 25
NSCH = NCHUNK // SCH
WROWS = N_NODES // NS

_BCAST_DNUMS = lax.GatherDimensionNumbers(
    offset_dims=(), collapsed_slice_dims=(0,), start_index_map=(0,))


def _bcast_lane(v, j):
  idx = jnp.full((L, 1), j, dtype=jnp.int32)
  return lax.gather(v, idx, _BCAST_DNUMS, (1,),
                    mode=lax.GatherScatterMode.PROMISE_IN_BOUNDS)


def _sc_body(x_hbm, sidx_hbm, tidx_hbm, en_hbm, es_hbm, out_hbm,
             acc_sh, sidx_v, tidx_v, en_v, es_v, rows_a, rows_b,
             gsem_a, gsem_b, ssem_a, ssem_b):
  cid = lax.axis_index("c")
  sid = lax.axis_index("s")
  wid = cid * NS + sid

  zvec = jnp.zeros((L,), jnp.float32)

  def _zrow(i, _):
    for k in range(D // L):
      rows_a[i, pl.ds(k * L, L)] = zvec
    return 0

  lax.fori_loop(0, CHUNK, _zrow, 0)
  for r in range(7):
    pltpu.sync_copy(rows_a, acc_sh.at[pl.ds(sid * WROWS + r * CHUNK, CHUNK)])
  pltpu.sync_copy(rows_a.at[pl.ds(0, WROWS - 7 * CHUNK)],
                  acc_sh.at[pl.ds(sid * WROWS + 7 * CHUNK,
                                  WROWS - 7 * CHUNK)])

  pltpu.sync_copy(sidx_hbm.at[wid], sidx_v)
  pltpu.sync_copy(tidx_hbm.at[wid], tidx_v)

  plsc.subcore_barrier()

  def _stage_w(c):
    @pl.when(lax.rem(c, SCH) == 0)
    def _():
      s = lax.div(c, SCH)
      pltpu.sync_copy(en_hbm.at[wid, s], en_v)
      pltpu.sync_copy(es_hbm.at[wid, s], es_v)

  def _gather_start(c, rows, sem):
    pltpu.async_copy(x_hbm.at[sidx_v.at[c]], rows, sem)

  def _gather_wait(c, rows, sem):
    pltpu.make_async_copy(x_hbm.at[sidx_v.at[c]], rows, sem).wait()

  def _scat_start(c, rows, sem):
    pltpu.async_copy(rows, acc_sh.at[tidx_v.at[c]], sem, add=True)

  def _scat_wait(c, rows, sem):
    pltpu.make_async_copy(rows, acc_sh.at[tidx_v.at[c]], sem).wait()

  def _scale(c, rows):
    c_l = lax.rem(c, SCH)

    @plsc.parallel_loop(0, CHUNK // L)
    def _group(g):
      w = en_v[c_l, pl.ds(g * L, L)] * es_v[c_l, pl.ds(g * L, L)]
      for j in range(L):
        e = g * L + j
        wj = _bcast_lane(w, j)
        for k in range(D // L):
          rows[e, pl.ds(k * L, L)] = rows[e, pl.ds(k * L, L)] * wj

  _gather_start(0, rows_a, gsem_a)

  def _pair(g, _):
    c0 = 2 * g
    c1 = c0 + 1
    _stage_w(c0)
    _gather_wait(c0, rows_a, gsem_a)

    @pl.when(g > 0)
    def _():
      _scat_wait(c1, rows_b, ssem_b)

    _gather_start(c1, rows_b, gsem_b)
    _scale(c0, rows_a)
    _scat_start(c0, rows_a, ssem_a)

    _stage_w(c1)
    _gather_wait(c1, rows_b, gsem_b)
    _scale(c1, rows_b)
    _scat_wait(c0, rows_a, ssem_a)
    _gather_start(c0 + 2, rows_a, gsem_a)
    _scat_start(c1, rows_b, ssem_b)
    return 0

  lax.fori_loop(0, (NCHUNK - 1) // 2, _pair, 0)

  ct = NCHUNK - 1
  _gather_wait(ct, rows_a, gsem_a)
  _scale(ct, rows_a)
  _scat_start(ct, rows_a, ssem_a)
  _scat_wait(ct, rows_a, ssem_a)
  _scat_wait(ct - 1, rows_b, ssem_b)

  plsc.subcore_barrier()

  for r in range(7):
    row0 = sid * WROWS + r * CHUNK
    pltpu.sync_copy(acc_sh.at[pl.ds(row0, CHUNK)], rows_a)
    pltpu.sync_copy(rows_a, out_hbm.at[cid, pl.ds(row0, CHUNK)])
  tail = WROWS - 7 * CHUNK
  row0 = sid * WROWS + 7 * CHUNK
  pltpu.sync_copy(acc_sh.at[pl.ds(row0, tail)], rows_a.at[pl.ds(0, tail)])
  pltpu.sync_copy(rows_a.at[pl.ds(0, tail)], out_hbm.at[cid, pl.ds(row0, tail)])


_sc_kernel = functools.partial(
    pl.kernel,
    out_type=jax.ShapeDtypeStruct((NC, N_NODES, D), jnp.float32),
    mesh=plsc.VectorSubcoreMesh(core_axis_name="c", subcore_axis_name="s"),
    compiler_params=pltpu.CompilerParams(use_tc_tiling_on_sc=False),
    scratch_types=[
        pltpu.VMEM_SHARED((N_NODES, D), jnp.float32),
        pltpu.VMEM((NCHUNK, CHUNK), jnp.int32),
        pltpu.VMEM((NCHUNK, CHUNK), jnp.int32),
        pltpu.VMEM((SCH, CHUNK), jnp.float32),
        pltpu.VMEM((SCH, CHUNK), jnp.float32),
        pltpu.VMEM((CHUNK, D), jnp.float32),
        pltpu.VMEM((CHUNK, D), jnp.float32),
        pltpu.SemaphoreType.DMA,
        pltpu.SemaphoreType.DMA,
        pltpu.SemaphoreType.DMA,
        pltpu.SemaphoreType.DMA,
    ],
)(_sc_body)


def _add_body(a_ref, o_ref):
  o_ref[...] = a_ref[0] + a_ref[1]


def _combine(partials):
  blk = N_NODES // 10
  return pl.pallas_call(
      _add_body,
      out_shape=jax.ShapeDtypeStruct((N_NODES, D), jnp.float32),
      grid=(N_NODES // blk,),
      in_specs=[pl.BlockSpec((NC, blk, D), lambda i: (0, i, 0))],
      out_specs=pl.BlockSpec((blk, D), lambda i: (i, 0)),
  )(partials)


def kernel(input, eidx, enorm, esgn):
  sidx = eidx[0].astype(jnp.int32).reshape(NW, NCHUNK, CHUNK)
  tidx = eidx[1].astype(jnp.int32).reshape(NW, NCHUNK, CHUNK)
  en = enorm.reshape(NW, NSCH, SCH, CHUNK)
  es = esgn.reshape(NW, NSCH, SCH, CHUNK)
  partials = _sc_kernel(input, sidx, tidx, en, es)
  return _combine(partials)


# speedup vs baseline: 9.7364x; 1.0544x over previous
import functools

import jax
import jax.numpy as jnp
from jax import lax
from jax.experimental import pallas as pl
from jax.experimental.pallas import tpu as pltpu
from jax.experimental.pallas import tpu_sc as plsc

N_NODES = 10000
N_EDGES = 320000
D = 128
L = 16

NC = 2
NS = 16
NW = NC * NS
EPW = N_EDGES // NW
CHUNK = 80
NCHUNK = EPW // CHUNK
SCH = 25
NSCH = NCHUNK // SCH
NBUF = 3
WROWS = N_NODES // NS

_BCAST_DNUMS = lax.GatherDimensionNumbers(
    offset_dims=(), collapsed_slice_dims=(0,), start_index_map=(0,))


def _bcast_lane(v, j):
  idx = jnp.full((L, 1), j, dtype=jnp.int32)
  return lax.gather(v, idx, _BCAST_DNUMS, (1,),
                    mode=lax.GatherScatterMode.PROMISE_IN_BOUNDS)


def _sc_body(x_hbm, sidx_hbm, tidx_hbm, en_hbm, es_hbm, out_hbm,
             acc_sh, sidx_v, tidx_v, en_v, es_v,
             rows_a, rows_b, rows_c,
             gsem_a, gsem_b, gsem_c, ssem_a, ssem_b, ssem_c):
  rows = (rows_a, rows_b, rows_c)
  gsems = (gsem_a, gsem_b, gsem_c)
  ssems = (ssem_a, ssem_b, ssem_c)

  cid = lax.axis_index("c")
  sid = lax.axis_index("s")
  wid = cid * NS + sid

  zvec = jnp.zeros((L,), jnp.float32)

  def _zrow(i, _):
    for k in range(D // L):
      rows[0][i, pl.ds(k * L, L)] = zvec
    return 0

  lax.fori_loop(0, CHUNK, _zrow, 0)
  for r in range(7):
    pltpu.sync_copy(rows[0], acc_sh.at[pl.ds(sid * WROWS + r * CHUNK, CHUNK)])
  pltpu.sync_copy(rows[0].at[pl.ds(0, WROWS - 7 * CHUNK)],
                  acc_sh.at[pl.ds(sid * WROWS + 7 * CHUNK,
                                  WROWS - 7 * CHUNK)])

  plsc.subcore_barrier()

  def _stage(s):
    pltpu.sync_copy(sidx_hbm.at[wid, s], sidx_v)
    pltpu.sync_copy(tidx_hbm.at[wid, s], tidx_v)
    pltpu.sync_copy(en_hbm.at[wid, s], en_v)
    pltpu.sync_copy(es_hbm.at[wid, s], es_v)

  def _gather_start(c, b):
    pltpu.async_copy(x_hbm.at[sidx_v.at[lax.rem(c, SCH)]], rows[b], gsems[b])

  def _gather_wait(c, b):
    pltpu.make_async_copy(x_hbm.at[sidx_v.at[lax.rem(c, SCH)]], rows[b],
                          gsems[b]).wait()

  def _scat_start(c, b):
    pltpu.async_copy(rows[b], acc_sh.at[tidx_v.at[lax.rem(c, SCH)]], ssems[b],
                     add=True)

  def _scat_wait(c, b):
    pltpu.make_async_copy(rows[b], acc_sh.at[tidx_v.at[lax.rem(c, SCH)]],
                          ssems[b]).wait()

  def _scale(c, b):
    c_l = lax.rem(c, SCH)

    @plsc.parallel_loop(0, CHUNK // L)
    def _group(g):
      w = en_v[c_l, pl.ds(g * L, L)] * es_v[c_l, pl.ds(g * L, L)]
      for j in range(L):
        e = g * L + j
        wj = _bcast_lane(w, j)
        for k in range(D // L):
          rows[b][e, pl.ds(k * L, L)] = rows[b][e, pl.ds(k * L, L)] * wj

  def _boundary(c, b):
    @pl.when(lax.rem(c, SCH) == 0)
    def _():
      @pl.when(c > 0)
      def _():
        _scat_wait(c - 2, (b + 1) % NBUF)
        _scat_wait(c - 1, (b + 2) % NBUF)
      _stage(lax.div(c, SCH))
      _gather_start(c, b)

  def _stepg(c, b):
    nxt = (b + 1) % NBUF

    @pl.when(jnp.logical_and(c >= 2, lax.rem(c, SCH) >= 2))
    def _():
      _scat_wait(c - 2, nxt)

    @pl.when(jnp.logical_and(c + 1 <= NCHUNK - 1, lax.rem(c + 1, SCH) != 0))
    def _():
      _gather_start(c + 1, nxt)

    _gather_wait(c, b)
    _scale(c, b)
    _scat_start(c, b)

  def _body(g, _):
    c = 3 * g
    _boundary(c, 0)
    _stepg(c, 0)
    _boundary(c + 1, 1)
    _stepg(c + 1, 1)
    _boundary(c + 2, 2)
    _stepg(c + 2, 2)
    return 0

  lax.fori_loop(0, 41, _body, 0)

  _scat_wait(121, 1)
  _gather_start(124, 1)
  _gather_wait(123, 0)
  _scale(123, 0)
  _scat_start(123, 0)
  _scat_wait(122, 2)
  _gather_wait(124, 1)
  _scale(124, 1)
  _scat_start(124, 1)
  _scat_wait(123, 0)
  _scat_wait(124, 1)

  plsc.subcore_barrier()

  for r in range(7):
    row0 = sid * WROWS + r * CHUNK
    pltpu.sync_copy(acc_sh.at[pl.ds(row0, CHUNK)], rows[0])
    pltpu.sync_copy(rows[0], out_hbm.at[cid, pl.ds(row0, CHUNK)])
  tail = WROWS - 7 * CHUNK
  row0 = sid * WROWS + 7 * CHUNK
  pltpu.sync_copy(acc_sh.at[pl.ds(row0, tail)], rows[0].at[pl.ds(0, tail)])
  pltpu.sync_copy(rows[0].at[pl.ds(0, tail)],
                  out_hbm.at[cid, pl.ds(row0, tail)])


_sc_kernel = functools.partial(
    pl.kernel,
    out_type=jax.ShapeDtypeStruct((NC, N_NODES, D), jnp.float32),
    mesh=plsc.VectorSubcoreMesh(core_axis_name="c", subcore_axis_name="s"),
    compiler_params=pltpu.CompilerParams(use_tc_tiling_on_sc=False),
    scratch_types=[
        pltpu.VMEM_SHARED((N_NODES, D), jnp.float32),
        pltpu.VMEM((SCH, CHUNK), jnp.int32),
        pltpu.VMEM((SCH, CHUNK), jnp.int32),
        pltpu.VMEM((SCH, CHUNK), jnp.float32),
        pltpu.VMEM((SCH, CHUNK), jnp.float32),
        pltpu.VMEM((CHUNK, D), jnp.float32),
        pltpu.VMEM((CHUNK, D), jnp.float32),
        pltpu.VMEM((CHUNK, D), jnp.float32),
        pltpu.SemaphoreType.DMA,
        pltpu.SemaphoreType.DMA,
        pltpu.SemaphoreType.DMA,
        pltpu.SemaphoreType.DMA,
        pltpu.SemaphoreType.DMA,
        pltpu.SemaphoreType.DMA,
    ],
)(_sc_body)


def _add_body(a_ref, o_ref):
  o_ref[...] = a_ref[0] + a_ref[1]


def _combine(partials):
  blk = N_NODES // 10
  return pl.pallas_call(
      _add_body,
      out_shape=jax.ShapeDtypeStruct((N_NODES, D), jnp.float32),
      grid=(N_NODES // blk,),
      in_specs=[pl.BlockSpec((NC, blk, D), lambda i: (0, i, 0))],
      out_specs=pl.BlockSpec((blk, D), lambda i: (i, 0)),
  )(partials)


def kernel(input, eidx, enorm, esgn):
  sidx = eidx[0].astype(jnp.int32).reshape(NW, NSCH, SCH, CHUNK)
  tidx = eidx[1].astype(jnp.int32).reshape(NW, NSCH, SCH, CHUNK)
  en = enorm.reshape(NW, NSCH, SCH, CHUNK)
  es = esgn.reshape(NW, NSCH, SCH, CHUNK)
  partials = _sc_kernel(input, sidx, tidx, en, es)
  return _combine(partials)
